# baseline (speedup 1.0000x reference)
import functools

import numpy as np

import jax
import jax.numpy as jnp
from jax import lax
from jax.experimental import pallas as pl
from jax.experimental.pallas import tpu as pltpu
from jax.experimental.pallas import tpu_sc as plsc

N = 10000
E = 320000
D = 128
H = 128
DE = 16
RBF_DIM = 20
R_CUT = 1.0
L = 4

NB = 5
BN = N // NB
EB = 4000
NEB = E // EB
NSUB = 16
HH = H // 2


def _silu(v):
    return v * jax.nn.sigmoid(v)


def _sc_radial(coords, src, dst):
    mesh = plsc.VectorSubcoreMesh(core_axis_name="c", subcore_axis_name="s")
    ept = E // (2 * NSUB)

    @functools.partial(
        pl.kernel,
        out_type=jax.ShapeDtypeStruct((E,), jnp.float32),
        mesh=mesh,
        compiler_params=pltpu.CompilerParams(needs_layout_passes=False),
        scratch_types=[
            pltpu.VMEM((N * 3,), jnp.float32),
            pltpu.VMEM((ept,), jnp.int32),
            pltpu.VMEM((ept,), jnp.int32),
            pltpu.VMEM((ept,), jnp.float32),
        ],
    )
    def k(coords_h, src_h, dst_h, out_h, cv, sv, dv, rv):
        c = lax.axis_index("c")
        s = lax.axis_index("s")
        base = (c * NSUB + s) * ept
        pltpu.sync_copy(coords_h, cv)
        pltpu.sync_copy(src_h.at[pl.ds(base, ept)], sv)
        pltpu.sync_copy(dst_h.at[pl.ds(base, ept)], dv)
        def body(j, carry):
            sl = pl.ds(j * 16, 16)
            si = sv[sl] * 3
            di = dv[sl] * 3
            dx = plsc.load_gather(cv, [si]) - plsc.load_gather(cv, [di])
            dy = plsc.load_gather(cv, [si + 1]) - plsc.load_gather(cv, [di + 1])
            dz = plsc.load_gather(cv, [si + 2]) - plsc.load_gather(cv, [di + 2])
            rv[sl] = dx * dx + dy * dy + dz * dz
            return carry

        lax.fori_loop(0, ept // 16, body, 0)
        pltpu.sync_copy(rv, out_h.at[pl.ds(base, ept)])

    return k(coords.reshape(N * 3), src, dst)


def _sc_scatter(l, h, gflat, src4, dst4, zeros128):
    mesh = plsc.VectorSubcoreMesh(core_axis_name="c", subcore_axis_name="s")
    ept = E // NSUB
    B = 80
    SB = 25
    NSC = ept // (SB * B)
    R0 = 632
    R15 = N - 15 * R0

    @functools.partial(
        pl.kernel,
        out_type=jax.ShapeDtypeStruct((2 * N, H), jnp.float32),
        mesh=mesh,
        compiler_params=pltpu.CompilerParams(needs_layout_passes=False),
        scratch_types=[
            pltpu.VMEM((SB, B), jnp.int32),
            pltpu.VMEM((SB, B), jnp.int32),
            pltpu.VMEM((B, H), jnp.float32),
            pltpu.VMEM((B, H), jnp.float32),
            pltpu.VMEM((B, H), jnp.float32),
            pltpu.VMEM((B * HH,), jnp.float32),
            pltpu.VMEM((B * HH,), jnp.float32),
            pltpu.VMEM_SHARED((N, H), jnp.float32),
            pltpu.SemaphoreType.DMA,
            pltpu.SemaphoreType.DMA,
            pltpu.SemaphoreType.DMA,
            pltpu.SemaphoreType.DMA,
        ],
    )
    def k(h_h, g_h, src_h, dst_h, z_h, out_h,
          sall, dall, rows0, rows1, pay0, gb0, gb1, acc,
          sg0, sg1, st0, st1):
        c = lax.axis_index("c")
        s = lax.axis_index("s")
        rows = [rows0, rows1]
        pay = [pay0, pay0]
        gbuf = [gb0, gb1]
        sg = [sg0, sg1]
        st = [st0, st1]

        @pl.when(s < 15)
        def _():
            pltpu.sync_copy(z_h.at[pl.ds(s * R0, R0)], acc.at[pl.ds(s * R0, R0)])

        @pl.when(s == 15)
        def _():
            pltpu.sync_copy(z_h.at[pl.ds(15 * R0, R15)], acc.at[pl.ds(15 * R0, R15)])

        plsc.subcore_barrier()
        gbase = (2 * l + c) * E + s * ept
        hoff = c * HH

        def super_chunk(k_, carry):
            gb = gbase + k_ * (SB * B)
            pltpu.sync_copy(src_h.at[s, k_], sall)
            pltpu.sync_copy(dst_h.at[s, k_], dall)

            def start(j, b):
                pltpu.async_copy(h_h.at[sall.at[j]], rows[b], sg[b])
                pltpu.async_copy(g_h.at[pl.ds((gb + j * B) * HH, B * HH)],
                                 gbuf[b], st[b])

            def handle(m, b):
                pltpu.make_async_copy(h_h.at[pl.ds(0, B)], rows[b], sg[b]).wait()
                pltpu.make_async_copy(g_h.at[pl.ds(0, B * HH)],
                                      gbuf[b], st[b]).wait()
                for r in range(B):
                    for j in range(HH // 16):
                        sl = pl.ds(j * 16, 16)
                        g16 = gbuf[b][pl.ds(r * HH + j * 16, 16)]
                        pay[b][r, sl] = g16 * rows[b][r, pl.ds(hoff + j * 16, 16)]
                        pay[b][r, pl.ds(HH + j * 16, 16)] = g16
                pltpu.sync_copy(pay[b], acc.at[dall.at[m]], add=True)

                @pl.when(m + 2 < SB)
                def _():
                    start(m + 2, b)

            start(0, 0)
            start(1, 1)

            def pair(m2, c2):
                handle(m2 * 2, 0)
                handle(m2 * 2 + 1, 1)
                return c2

            lax.fori_loop(0, SB // 2, pair, 0)
            handle(SB - 1, 0)
            return carry

        lax.fori_loop(0, NSC, super_chunk, 0)
        plsc.subcore_barrier()

        @pl.when(s < 15)
        def _():
            pltpu.sync_copy(acc.at[pl.ds(s * R0, R0)],
                            out_h.at[pl.ds(c * N + s * R0, R0)])

        @pl.when(s == 15)
        def _():
            pltpu.sync_copy(acc.at[pl.ds(15 * R0, R15)],
                            out_h.at[pl.ds(c * N + 15 * R0, R15)])

    return k(h, gflat, src4, dst4, zeros128)


def _tc_inconv(x, Win1, bin1, Win2, bin2):
    def body(x_ref, w1_ref, b1_ref, w2_ref, b2_ref, h_ref):
        t = _silu(jnp.dot(x_ref[...], w1_ref[...],
                          preferred_element_type=jnp.float32) + b1_ref[...])
        h_ref[...] = jnp.dot(t, w2_ref[...],
                             preferred_element_type=jnp.float32) + b2_ref[...]

    return pl.pallas_call(
        body,
        grid=(NB,),
        in_specs=[
            pl.BlockSpec((BN, D), lambda i: (i, 0)),
            pl.BlockSpec((D, H), lambda i: (0, 0)),
            pl.BlockSpec((1, H), lambda i: (0, 0)),
            pl.BlockSpec((H, H), lambda i: (0, 0)),
            pl.BlockSpec((1, H), lambda i: (0, 0)),
        ],
        out_specs=pl.BlockSpec((BN, H), lambda i: (i, 0)),
        out_shape=jax.ShapeDtypeStruct((N, H), jnp.float32),
    )(x, Win1, bin1, Win2, bin2)


def _tc_gates(edge_attr, radial2, mu_row, We, be):
    def body(ea_ref, rad_ref, mu_ref, we_ref, be_ref, g_ref):
        normv = jnp.sqrt(rad_ref[...]) + 1e-8
        gamma = RBF_DIM / R_CUT
        rbf = jnp.exp(-gamma * (normv - mu_ref[...]) ** 2)
        ef = jnp.concatenate([ea_ref[...], rbf], axis=1)
        g = jnp.dot(ef, we_ref[0], preferred_element_type=jnp.float32) + be_ref[0]
        g = _silu(g)
        g_ref[0, 0] = g[:, :HH]
        g_ref[0, 1] = g[:, HH:]

    return pl.pallas_call(
        body,
        grid=(L, NEB),
        in_specs=[
            pl.BlockSpec((EB, DE), lambda l, i: (i, 0)),
            pl.BlockSpec((EB, 1), lambda l, i: (i, 0)),
            pl.BlockSpec((1, RBF_DIM), lambda l, i: (0, 0)),
            pl.BlockSpec((1, DE + RBF_DIM, H), lambda l, i: (l, 0, 0)),
            pl.BlockSpec((1, 1, H), lambda l, i: (l, 0, 0)),
        ],
        out_specs=pl.BlockSpec((1, 2, EB, HH), lambda l, i: (l, 0, i, 0)),
        out_shape=jax.ShapeDtypeStruct((L, 2, E, HH), jnp.float32),
    )(edge_attr, radial2, mu_row, We, be.reshape(L, 1, H))


def _tc_layer_a(l, h, SG, Wu, bu, seg2):
    def body(h_ref, sg_ref, wu_ref, bu_ref, seg_ref,
             hi_ref, ls_ref, rs_ref, cn_ref):
        hv = h_ref[...]
        sg0 = sg_ref[0]
        sg1 = sg_ref[1]
        agg = jnp.concatenate(
            [sg0[:, :HH] + hv[:, :HH] * sg0[:, HH:],
             sg1[:, :HH] + hv[:, HH:] * sg1[:, HH:]],
            axis=1)
        u = _silu(jnp.dot(agg, wu_ref[0], preferred_element_type=jnp.float32)
                  + bu_ref[0])
        hi = hv + u
        hi_ref[...] = hi
        lm = (seg_ref[...] == 1).astype(jnp.float32)
        rm = 1.0 - lm
        ls_ref[0] = jnp.sum(hi * lm, axis=0, keepdims=True)
        rs_ref[0] = jnp.sum(hi * rm, axis=0, keepdims=True)
        lane = lax.broadcasted_iota(jnp.int32, (1, H), 1)
        cl = jnp.sum(lm)
        cr = jnp.sum(rm)
        cn_ref[0] = jnp.where(lane == 0, cl, jnp.where(lane == 1, cr, 0.0))

    return pl.pallas_call(
        body,
        grid=(NB,),
        in_specs=[
            pl.BlockSpec((BN, H), lambda i: (i, 0)),
            pl.BlockSpec((2, BN, H), lambda i: (0, i, 0)),
            pl.BlockSpec((1, H, H), lambda i, l=l: (l, 0, 0)),
            pl.BlockSpec((1, 1, H), lambda i, l=l: (l, 0, 0)),
            pl.BlockSpec((BN, 1), lambda i: (i, 0)),
        ],
        out_specs=[
            pl.BlockSpec((BN, H), lambda i: (i, 0)),
            pl.BlockSpec((1, 1, H), lambda i: (i, 0, 0)),
            pl.BlockSpec((1, 1, H), lambda i: (i, 0, 0)),
            pl.BlockSpec((1, 1, H), lambda i: (i, 0, 0)),
        ],
        out_shape=[
            jax.ShapeDtypeStruct((N, H), jnp.float32),
            jax.ShapeDtypeStruct((NB, 1, H), jnp.float32),
            jax.ShapeDtypeStruct((NB, 1, H), jnp.float32),
            jax.ShapeDtypeStruct((NB, 1, H), jnp.float32),
        ],
    )(h, SG, Wu, bu.reshape(L, 1, H), seg2)


def _tc_layer_b(l, hi, ls, rs, cn, A, Wi1, bi1, Wi2, bi2, seg2):
    def body(h_ref, ls_ref, rs_ref, cn_ref, seg_ref,
             a_ref, w1_ref, b1_ref, w2_ref, b2_ref, hn_ref):
        cnv = cn_ref[...][:, 0, :]
        nl = jnp.maximum(jnp.sum(cnv[:, 0]), 1.0)
        nr = jnp.maximum(jnp.sum(cnv[:, 1]), 1.0)
        ml = jnp.sum(ls_ref[...][:, 0, :], axis=0) / nl
        mr = jnp.sum(rs_ref[...][:, 0, :], axis=0) / nr
        av = a_ref[0]
        aml = jnp.sum(av * ml[None, :], axis=1)[None, :]
        amr = jnp.sum(av * mr[None, :], axis=1)[None, :]
        hv = h_ref[...]
        sc_l = jnp.sum(hv * aml, axis=1, keepdims=True)
        sc_r = jnp.sum(hv * amr, axis=1, keepdims=True)
        gate = jax.nn.sigmoid(jnp.where(seg_ref[...] == 0, sc_l, sc_r))
        t = _silu(jnp.dot(hv, w1_ref[0], preferred_element_type=jnp.float32)
                  + b1_ref[0])
        inter = jnp.dot(t, w2_ref[0], preferred_element_type=jnp.float32) + b2_ref[0]
        hn_ref[...] = hv + gate * inter

    return pl.pallas_call(
        body,
        grid=(NB,),
        in_specs=[
            pl.BlockSpec((BN, H), lambda i: (i, 0)),
            pl.BlockSpec((NB, 1, H), lambda i: (0, 0, 0)),
            pl.BlockSpec((NB, 1, H), lambda i: (0, 0, 0)),
            pl.BlockSpec((NB, 1, H), lambda i: (0, 0, 0)),
            pl.BlockSpec((BN, 1), lambda i: (i, 0)),
            pl.BlockSpec((1, H, H), lambda i, l=l: (l, 0, 0)),
            pl.BlockSpec((1, H, H), lambda i, l=l: (l, 0, 0)),
            pl.BlockSpec((1, 1, H), lambda i, l=l: (l, 0, 0)),
            pl.BlockSpec((1, H, H), lambda i, l=l: (l, 0, 0)),
            pl.BlockSpec((1, 1, H), lambda i, l=l: (l, 0, 0)),
        ],
        out_specs=pl.BlockSpec((BN, H), lambda i: (i, 0)),
        out_shape=jax.ShapeDtypeStruct((N, H), jnp.float32),
    )(hi, ls, rs, cn, seg2, A, Wi1, bi1.reshape(L, 1, H), Wi2, bi2.reshape(L, 1, H))


def kernel(x, coords, edge_attr, Win1, bin1, Win2, bin2, We, be, Wu, bu,
           A, Wi1, bi1, Wi2, bi2, edge_index, seg):
    src = edge_index[0].astype(jnp.int32)
    dst = edge_index[1].astype(jnp.int32)
    seg2 = seg.reshape(N, 1).astype(jnp.int32)
    radial = _sc_radial(coords, src, dst)
    src4 = src.reshape(NSUB, 10, 25, 80)
    dst4 = dst.reshape(NSUB, 10, 25, 80)
    h = _tc_inconv(x, Win1, bin1.reshape(1, H), Win2, bin2.reshape(1, H))
    mu_row = jnp.asarray(np.linspace(0.0, R_CUT, RBF_DIM,
                                     dtype=np.float32).reshape(1, RBF_DIM))
    gflat = _tc_gates(edge_attr, radial.reshape(E, 1), mu_row, We,
                      be).reshape(L * 2 * E * HH)
    z128 = jnp.zeros((N, H), jnp.float32)
    for l in range(L):
        SG = _sc_scatter(l, h, gflat, src4, dst4, z128)
        hi, ls, rs, cn = _tc_layer_a(l, h, SG.reshape(2, N, H), Wu, bu, seg2)
        h = _tc_layer_b(l, hi, ls, rs, cn, A, Wi1, bi1, Wi2, bi2, seg2)
    return h


# speedup vs baseline: 1.5953x; 1.5953x over previous
import functools

import numpy as np

import jax
import jax.numpy as jnp
from jax import lax
from jax.experimental import pallas as pl
from jax.experimental.pallas import tpu as pltpu
from jax.experimental.pallas import tpu_sc as plsc

N = 10000
E = 320000
D = 128
H = 128
DE = 16
RBF_DIM = 20
R_CUT = 1.0
L = 4

NB = 5
BN = N // NB
EB = 4000
NEB = E // EB
NSUB = 16
HH = H // 2


def _silu(v):
    return v * jax.nn.sigmoid(v)


def _sc_radial(coords, src, dst):
    mesh = plsc.VectorSubcoreMesh(core_axis_name="c", subcore_axis_name="s")
    ept = E // (2 * NSUB)

    @functools.partial(
        pl.kernel,
        out_type=jax.ShapeDtypeStruct((E,), jnp.float32),
        mesh=mesh,
        compiler_params=pltpu.CompilerParams(needs_layout_passes=False),
        scratch_types=[
            pltpu.VMEM((N * 3,), jnp.float32),
            pltpu.VMEM((ept,), jnp.int32),
            pltpu.VMEM((ept,), jnp.int32),
            pltpu.VMEM((ept,), jnp.float32),
        ],
    )
    def k(coords_h, src_h, dst_h, out_h, cv, sv, dv, rv):
        c = lax.axis_index("c")
        s = lax.axis_index("s")
        base = (c * NSUB + s) * ept
        pltpu.sync_copy(coords_h, cv)
        pltpu.sync_copy(src_h.at[pl.ds(base, ept)], sv)
        pltpu.sync_copy(dst_h.at[pl.ds(base, ept)], dv)
        def body(j, carry):
            sl = pl.ds(j * 16, 16)
            si = sv[sl] * 3
            di = dv[sl] * 3
            dx = plsc.load_gather(cv, [si]) - plsc.load_gather(cv, [di])
            dy = plsc.load_gather(cv, [si + 1]) - plsc.load_gather(cv, [di + 1])
            dz = plsc.load_gather(cv, [si + 2]) - plsc.load_gather(cv, [di + 2])
            rv[sl] = dx * dx + dy * dy + dz * dz
            return carry

        lax.fori_loop(0, ept // 16, body, 0)
        pltpu.sync_copy(rv, out_h.at[pl.ds(base, ept)])

    return k(coords.reshape(N * 3), src, dst)


def _sc_scatter(h, gflat, src4, dst4, zeros128):
    mesh = plsc.VectorSubcoreMesh(core_axis_name="c", subcore_axis_name="s")
    ept = E // NSUB
    B = 80
    SB = 25
    NSC = ept // (SB * B)
    R0 = 632
    R15 = N - 15 * R0

    @functools.partial(
        pl.kernel,
        out_type=jax.ShapeDtypeStruct((2 * N, H), jnp.float32),
        mesh=mesh,
        compiler_params=pltpu.CompilerParams(needs_layout_passes=False),
        scratch_types=[
            pltpu.VMEM((SB, B), jnp.int32),
            pltpu.VMEM((SB, B), jnp.int32),
            pltpu.VMEM((B, H), jnp.float32),
            pltpu.VMEM((B, H), jnp.float32),
            pltpu.VMEM((B, H), jnp.float32),
            pltpu.VMEM((B, H), jnp.float32),
            pltpu.VMEM_SHARED((N, H), jnp.float32),
            pltpu.SemaphoreType.DMA,
            pltpu.SemaphoreType.DMA,
            pltpu.SemaphoreType.DMA,
            pltpu.SemaphoreType.DMA,
        ],
    )
    def k(h_h, g_h, src_h, dst_h, z_h, out_h,
          sall, dall, rows0, rows1, pay0, pay1, acc,
          sg0, sg1, st0, st1):
        c = lax.axis_index("c")
        s = lax.axis_index("s")
        rows = [rows0, rows1]
        pay = [pay0, pay1]
        sg = [sg0, sg1]
        st = [st0, st1]

        @pl.when(s < 15)
        def _():
            pltpu.sync_copy(z_h.at[pl.ds(s * R0, R0)], acc.at[pl.ds(s * R0, R0)])

        @pl.when(s == 15)
        def _():
            pltpu.sync_copy(z_h.at[pl.ds(15 * R0, R15)], acc.at[pl.ds(15 * R0, R15)])

        plsc.subcore_barrier()
        gbase = c * E + s * ept
        hoff = c * HH

        def super_chunk(k_, carry):
            gb = gbase + k_ * (SB * B)
            pltpu.sync_copy(src_h.at[s, k_], sall)
            pltpu.sync_copy(dst_h.at[s, k_], dall)

            def start(j, b):
                pltpu.async_copy(h_h.at[sall.at[j]], rows[b], sg[b])
                pltpu.async_copy(g_h.at[pl.ds(gb + j * B, B)], pay[b], st[b])

            def handle(m, b):
                pltpu.make_async_copy(h_h.at[pl.ds(0, B)], rows[b], sg[b]).wait()
                pltpu.make_async_copy(h_h.at[pl.ds(0, B)], pay[b], st[b]).wait()
                for r in range(B):
                    for j in range(HH // 16):
                        sl = pl.ds(j * 16, 16)
                        pay[b][r, sl] = (pay[b][r, sl]
                                         * rows[b][r, pl.ds(hoff + j * 16, 16)])
                pltpu.sync_copy(pay[b], acc.at[dall.at[m]], add=True)

                @pl.when(m + 2 < SB)
                def _():
                    start(m + 2, b)

            start(0, 0)
            start(1, 1)

            def pair(m2, c2):
                handle(m2 * 2, 0)
                handle(m2 * 2 + 1, 1)
                return c2

            lax.fori_loop(0, SB // 2, pair, 0)
            handle(SB - 1, 0)
            return carry

        lax.fori_loop(0, NSC, super_chunk, 0)
        plsc.subcore_barrier()

        @pl.when(s < 15)
        def _():
            pltpu.sync_copy(acc.at[pl.ds(s * R0, R0)],
                            out_h.at[pl.ds(c * N + s * R0, R0)])

        @pl.when(s == 15)
        def _():
            pltpu.sync_copy(acc.at[pl.ds(15 * R0, R15)],
                            out_h.at[pl.ds(c * N + 15 * R0, R15)])

    return k(h, gflat, src4, dst4, zeros128)


def _tc_inconv(x, Win1, bin1, Win2, bin2):
    def body(x_ref, w1_ref, b1_ref, w2_ref, b2_ref, h_ref):
        t = _silu(jnp.dot(x_ref[...], w1_ref[...],
                          preferred_element_type=jnp.float32) + b1_ref[...])
        h_ref[...] = jnp.dot(t, w2_ref[...],
                             preferred_element_type=jnp.float32) + b2_ref[...]

    return pl.pallas_call(
        body,
        grid=(NB,),
        in_specs=[
            pl.BlockSpec((BN, D), lambda i: (i, 0)),
            pl.BlockSpec((D, H), lambda i: (0, 0)),
            pl.BlockSpec((1, H), lambda i: (0, 0)),
            pl.BlockSpec((H, H), lambda i: (0, 0)),
            pl.BlockSpec((1, H), lambda i: (0, 0)),
        ],
        out_specs=pl.BlockSpec((BN, H), lambda i: (i, 0)),
        out_shape=jax.ShapeDtypeStruct((N, H), jnp.float32),
    )(x, Win1, bin1, Win2, bin2)


def _tc_gates(l, edge_attr, radial2, mu_row, We, be):
    def body(ea_ref, rad_ref, mu_ref, we_ref, be_ref, g_ref):
        normv = jnp.sqrt(rad_ref[...]) + 1e-8
        gamma = RBF_DIM / R_CUT
        rbf = jnp.exp(-gamma * (normv - mu_ref[...]) ** 2)
        ef = jnp.concatenate([ea_ref[...], rbf], axis=1)
        g = jnp.dot(ef, we_ref[0], preferred_element_type=jnp.float32) + be_ref[0]
        g = _silu(g)
        g_ref[0] = jnp.concatenate([g[:, :HH], g[:, :HH]], axis=1)
        g_ref[1] = jnp.concatenate([g[:, HH:], g[:, HH:]], axis=1)

    return pl.pallas_call(
        body,
        grid=(NEB,),
        in_specs=[
            pl.BlockSpec((EB, DE), lambda i: (i, 0)),
            pl.BlockSpec((EB, 1), lambda i: (i, 0)),
            pl.BlockSpec((1, RBF_DIM), lambda i: (0, 0)),
            pl.BlockSpec((1, DE + RBF_DIM, H), lambda i, l=l: (l, 0, 0)),
            pl.BlockSpec((1, 1, H), lambda i, l=l: (l, 0, 0)),
        ],
        out_specs=pl.BlockSpec((2, EB, H), lambda i: (0, i, 0)),
        out_shape=jax.ShapeDtypeStruct((2, E, H), jnp.float32),
    )(edge_attr, radial2, mu_row, We, be.reshape(L, 1, H))


def _tc_layer_a(l, h, SG, Wu, bu, seg2):
    def body(h_ref, sg_ref, wu_ref, bu_ref, seg_ref,
             hi_ref, ls_ref, rs_ref, cn_ref):
        hv = h_ref[...]
        sg0 = sg_ref[0]
        sg1 = sg_ref[1]
        agg = jnp.concatenate(
            [sg0[:, :HH] + hv[:, :HH] * sg0[:, HH:],
             sg1[:, :HH] + hv[:, HH:] * sg1[:, HH:]],
            axis=1)
        u = _silu(jnp.dot(agg, wu_ref[0], preferred_element_type=jnp.float32)
                  + bu_ref[0])
        hi = hv + u
        hi_ref[...] = hi
        lm = (seg_ref[...] == 1).astype(jnp.float32)
        rm = 1.0 - lm
        ls_ref[0] = jnp.sum(hi * lm, axis=0, keepdims=True)
        rs_ref[0] = jnp.sum(hi * rm, axis=0, keepdims=True)
        lane = lax.broadcasted_iota(jnp.int32, (1, H), 1)
        cl = jnp.sum(lm)
        cr = jnp.sum(rm)
        cn_ref[0] = jnp.where(lane == 0, cl, jnp.where(lane == 1, cr, 0.0))

    return pl.pallas_call(
        body,
        grid=(NB,),
        in_specs=[
            pl.BlockSpec((BN, H), lambda i: (i, 0)),
            pl.BlockSpec((2, BN, H), lambda i: (0, i, 0)),
            pl.BlockSpec((1, H, H), lambda i, l=l: (l, 0, 0)),
            pl.BlockSpec((1, 1, H), lambda i, l=l: (l, 0, 0)),
            pl.BlockSpec((BN, 1), lambda i: (i, 0)),
        ],
        out_specs=[
            pl.BlockSpec((BN, H), lambda i: (i, 0)),
            pl.BlockSpec((1, 1, H), lambda i: (i, 0, 0)),
            pl.BlockSpec((1, 1, H), lambda i: (i, 0, 0)),
            pl.BlockSpec((1, 1, H), lambda i: (i, 0, 0)),
        ],
        out_shape=[
            jax.ShapeDtypeStruct((N, H), jnp.float32),
            jax.ShapeDtypeStruct((NB, 1, H), jnp.float32),
            jax.ShapeDtypeStruct((NB, 1, H), jnp.float32),
            jax.ShapeDtypeStruct((NB, 1, H), jnp.float32),
        ],
    )(h, SG, Wu, bu.reshape(L, 1, H), seg2)


def _tc_layer_b(l, hi, ls, rs, cn, A, Wi1, bi1, Wi2, bi2, seg2):
    def body(h_ref, ls_ref, rs_ref, cn_ref, seg_ref,
             a_ref, w1_ref, b1_ref, w2_ref, b2_ref, hn_ref):
        cnv = cn_ref[...][:, 0, :]
        nl = jnp.maximum(jnp.sum(cnv[:, 0]), 1.0)
        nr = jnp.maximum(jnp.sum(cnv[:, 1]), 1.0)
        ml = jnp.sum(ls_ref[...][:, 0, :], axis=0) / nl
        mr = jnp.sum(rs_ref[...][:, 0, :], axis=0) / nr
        av = a_ref[0]
        aml = jnp.sum(av * ml[None, :], axis=1)[None, :]
        amr = jnp.sum(av * mr[None, :], axis=1)[None, :]
        hv = h_ref[...]
        sc_l = jnp.sum(hv * aml, axis=1, keepdims=True)
        sc_r = jnp.sum(hv * amr, axis=1, keepdims=True)
        gate = jax.nn.sigmoid(jnp.where(seg_ref[...] == 0, sc_l, sc_r))
        t = _silu(jnp.dot(hv, w1_ref[0], preferred_element_type=jnp.float32)
                  + b1_ref[0])
        inter = jnp.dot(t, w2_ref[0], preferred_element_type=jnp.float32) + b2_ref[0]
        hn_ref[...] = hv + gate * inter

    return pl.pallas_call(
        body,
        grid=(NB,),
        in_specs=[
            pl.BlockSpec((BN, H), lambda i: (i, 0)),
            pl.BlockSpec((NB, 1, H), lambda i: (0, 0, 0)),
            pl.BlockSpec((NB, 1, H), lambda i: (0, 0, 0)),
            pl.BlockSpec((NB, 1, H), lambda i: (0, 0, 0)),
            pl.BlockSpec((BN, 1), lambda i: (i, 0)),
            pl.BlockSpec((1, H, H), lambda i, l=l: (l, 0, 0)),
            pl.BlockSpec((1, H, H), lambda i, l=l: (l, 0, 0)),
            pl.BlockSpec((1, 1, H), lambda i, l=l: (l, 0, 0)),
            pl.BlockSpec((1, H, H), lambda i, l=l: (l, 0, 0)),
            pl.BlockSpec((1, 1, H), lambda i, l=l: (l, 0, 0)),
        ],
        out_specs=pl.BlockSpec((BN, H), lambda i: (i, 0)),
        out_shape=jax.ShapeDtypeStruct((N, H), jnp.float32),
    )(hi, ls, rs, cn, seg2, A, Wi1, bi1.reshape(L, 1, H), Wi2, bi2.reshape(L, 1, H))


def kernel(x, coords, edge_attr, Win1, bin1, Win2, bin2, We, be, Wu, bu,
           A, Wi1, bi1, Wi2, bi2, edge_index, seg):
    src = edge_index[0].astype(jnp.int32)
    dst = edge_index[1].astype(jnp.int32)
    seg2 = seg.reshape(N, 1).astype(jnp.int32)
    radial = _sc_radial(coords, src, dst)
    src4 = src.reshape(NSUB, 10, 25, 80)
    dst4 = dst.reshape(NSUB, 10, 25, 80)
    h = _tc_inconv(x, Win1, bin1.reshape(1, H), Win2, bin2.reshape(1, H))
    mu_row = jnp.asarray(np.linspace(0.0, R_CUT, RBF_DIM,
                                     dtype=np.float32).reshape(1, RBF_DIM))
    gates = [_tc_gates(l, edge_attr, radial.reshape(E, 1), mu_row, We,
                       be).reshape(2 * E, H) for l in range(L)]
    z128 = jnp.zeros((N, H), jnp.float32)
    for l in range(L):
        SG = _sc_scatter(h, gates[l], src4, dst4, z128)
        hi, ls, rs, cn = _tc_layer_a(l, h, SG.reshape(2, N, H), Wu, bu, seg2)
        h = _tc_layer_b(l, hi, ls, rs, cn, A, Wi1, bi1, Wi2, bi2, seg2)
    return h
